# Initial kernel scaffold; baseline (speedup 1.0000x reference)
import jax
import jax.numpy as jnp
from jax.experimental import pallas as pl


def kernel(x, batch):
    raise NotImplementedError("write your pallas kernel here")


# speedup vs baseline: 7.6704x; 7.6704x over previous
import functools

import jax
import jax.numpy as jnp
from jax import lax
from jax.experimental import pallas as pl
from jax.experimental.pallas import tpu as pltpu
from jax.experimental.pallas import tpu_sc as plsc

NUM_SEGS = 128
LANES = 16
CHUNK = 128


@functools.cache
def _make_sc_kernel(N, H, S, C):
    info = plsc.get_sparse_core_info()
    NW = info.num_cores * info.num_subcores
    assert S % NW == 0 and H % LANES == 0 and N % LANES == 0
    SPW = S // NW
    F = H // LANES
    CSH = C.bit_length() - 1
    mesh = plsc.VectorSubcoreMesh(core_axis_name="c", subcore_axis_name="s")

    @functools.partial(
        pl.kernel,
        out_type=jax.ShapeDtypeStruct((S * 2 * H,), jnp.float32),
        mesh=mesh,
        compiler_params=pltpu.CompilerParams(needs_layout_passes=False),
        scratch_types=[
            pltpu.VMEM((N,), jnp.int32),
            pltpu.VMEM((C * H,), jnp.float32),
            pltpu.VMEM((2 * H,), jnp.float32),
        ],
    )
    def k(x_hbm, batch_hbm, out_hbm, batch_v, buf_v, orow_v):
        wid = lax.axis_index("s") * info.num_cores + lax.axis_index("c")
        seg0 = wid * SPW
        pltpu.sync_copy(batch_hbm, batch_v)

        targets = seg0 + lax.iota(jnp.int32, LANES)
        lo = jnp.zeros((LANES,), jnp.int32)
        hi = jnp.full((LANES,), N, jnp.int32)

        def bs_body(_, lh):
            lo, hi = lh
            mid = lax.shift_right_logical(lo + hi, 1)
            vals = plsc.load_gather(batch_v, [mid])
            pred = vals < targets
            return jnp.where(pred, mid + 1, lo), jnp.where(pred, hi, mid)

        bounds, _ = lax.fori_loop(0, 17, bs_body, (lo, hi))

        for kseg in range(SPW):
            r0 = bounds[kseg]
            r1 = bounds[kseg + 1]
            cnt = r1 - r0
            nchunks = lax.shift_right_logical(cnt + (C - 1), CSH)
            sums = (jnp.zeros((LANES,), jnp.float32),) * F
            maxs = (jnp.full((LANES,), -jnp.inf, jnp.float32),) * F

            def chunk_body(i, carry, r0=r0, r1=r1):
                base = r0 + i * C
                base_cl = jnp.minimum(base, N - C)
                off = base - base_cl
                pltpu.sync_copy(x_hbm.at[pl.ds(base_cl * H, C * H)], buf_v)
                nrows = jnp.minimum(C, r1 - base)

                def row_body(j, car, off=off):
                    s, m = car
                    rb = (j + off) * H
                    new_s, new_m = [], []
                    for f in range(F):
                        v = buf_v[pl.ds(rb + f * LANES, LANES)]
                        new_s.append(s[f] + v)
                        new_m.append(jnp.maximum(m[f], v))
                    return tuple(new_s), tuple(new_m)

                return lax.fori_loop(0, nrows, row_body, carry)

            sums, maxs = lax.fori_loop(0, nchunks, chunk_body, (sums, maxs))

            cntv = lax.broadcast_in_dim(cnt, (LANES,), ())
            scale = 1.0 / jnp.maximum(cntv.astype(jnp.float32), 1.0)
            nonempty = cntv > 0
            for f in range(F):
                orow_v[pl.ds(f * LANES, LANES)] = sums[f] * scale
                orow_v[pl.ds(H + f * LANES, LANES)] = jnp.where(
                    nonempty, maxs[f], 0.0
                )
            pltpu.sync_copy(
                orow_v, out_hbm.at[pl.ds((seg0 + kseg) * 2 * H, 2 * H)]
            )

    return k


def kernel(x, batch):
    N, H = x.shape
    out = _make_sc_kernel(N, H, NUM_SEGS, CHUNK)(x.reshape(-1), batch)
    return out.reshape(NUM_SEGS, 2 * H)


# speedup vs baseline: 11.6230x; 1.5153x over previous
import functools

import jax
import jax.numpy as jnp
from jax import lax
from jax.experimental import pallas as pl
from jax.experimental.pallas import tpu as pltpu
from jax.experimental.pallas import tpu_sc as plsc

NUM_SEGS = 128
LANES = 16
CHUNK = 128
_FMIN = float(jnp.finfo(jnp.float32).min)


@functools.cache
def _make_sc_kernel(N, H, S, C):
    info = plsc.get_sparse_core_info()
    NW = info.num_cores * info.num_subcores
    assert S % NW == 0 and H % LANES == 0 and N % LANES == 0
    SPW = S // NW
    F = H // LANES
    CSH = C.bit_length() - 1
    mesh = plsc.VectorSubcoreMesh(core_axis_name="c", subcore_axis_name="s")

    @functools.partial(
        pl.kernel,
        out_type=jax.ShapeDtypeStruct((S * 2 * H,), jnp.float32),
        mesh=mesh,
        scratch_types=[
            pltpu.VMEM((N + LANES,), jnp.int32),
            pltpu.VMEM((C, H), jnp.float32),
            pltpu.VMEM((2 * H,), jnp.float32),
        ],
    )
    def k(x_hbm, batch_hbm, out_hbm, batch_v, buf_v, orow_v):
        wid = lax.axis_index("s") * info.num_cores + lax.axis_index("c")
        seg0 = wid * SPW
        pltpu.sync_copy(batch_hbm, batch_v.at[pl.ds(0, N)])

        def lower_bound(t):
            def bs_body(_, lh):
                lo, hi = lh
                mid = lax.shift_right_logical(lo + hi, 1)
                val = batch_v[pl.ds(mid, LANES)][0]
                pred = val < t
                return jnp.where(pred, mid + 1, lo), jnp.where(pred, hi, mid)

            lo, _ = lax.fori_loop(
                0, 17, bs_body, (jnp.int32(0), jnp.int32(N))
            )
            return lo

        bounds = [lower_bound(seg0 + kk) for kk in range(SPW + 1)]

        for kseg in range(SPW):
            r0 = bounds[kseg]
            r1 = bounds[kseg + 1]
            cnt = r1 - r0
            base_a = lax.bitwise_and(r0, jnp.int32(~7))
            nchunks = lax.shift_right_logical(r1 - base_a + (C - 1), CSH)
            sums = (jnp.zeros((LANES,), jnp.float32),) * F
            maxs = (jnp.full((LANES,), _FMIN, jnp.float32),) * F

            def chunk_body(i, carry, r0=r0, r1=r1, base_a=base_a):
                start = base_a + i * C
                start_cl = pl.multiple_of(jnp.minimum(start, N - C), 8)
                sh = start - start_cl
                pltpu.sync_copy(x_hbm.at[pl.ds(start_cl, C)], buf_v)
                jlo = jnp.maximum(r0 - start, 0) + sh
                jhi = jnp.minimum(C, r1 - start) + sh

                def row_body(j, car):
                    s, m = car
                    new_s, new_m = [], []
                    for f in range(F):
                        v = buf_v[j, pl.ds(f * LANES, LANES)]
                        new_s.append(s[f] + v)
                        new_m.append(jnp.maximum(m[f], v))
                    return tuple(new_s), tuple(new_m)

                return lax.fori_loop(jlo, jhi, row_body, carry)

            sums, maxs = lax.fori_loop(0, nchunks, chunk_body, (sums, maxs))

            cntf = lax.broadcast_in_dim(cnt, (LANES,), ()).astype(jnp.float32)
            scale = 1.0 / jnp.maximum(cntf, 1.0)
            nonempty = jnp.minimum(cntf, 1.0)
            for f in range(F):
                orow_v[pl.ds(f * LANES, LANES)] = sums[f] * scale
                orow_v[pl.ds(H + f * LANES, LANES)] = maxs[f] * nonempty
            pltpu.sync_copy(
                orow_v, out_hbm.at[pl.ds((seg0 + kseg) * 2 * H, 2 * H)]
            )

    return k


def kernel(x, batch):
    N, H = x.shape
    out = _make_sc_kernel(N, H, NUM_SEGS, CHUNK)(x, batch)
    return out.reshape(NUM_SEGS, 2 * H)


# speedup vs baseline: 13.4245x; 1.1550x over previous
import functools

import jax
import jax.numpy as jnp
from jax import lax
from jax.experimental import pallas as pl
from jax.experimental.pallas import tpu as pltpu
from jax.experimental.pallas import tpu_sc as plsc

NUM_SEGS = 128
LANES = 16
CHUNK = 128
_FMIN = float(jnp.finfo(jnp.float32).min)


@functools.cache
def _make_sc_kernel(N, H, S, C):
    info = plsc.get_sparse_core_info()
    NW = info.num_cores * info.num_subcores
    assert S % NW == 0 and H % LANES == 0 and N % LANES == 0
    SPW = S // NW
    F = H // LANES
    CSH = C.bit_length() - 1
    mesh = plsc.VectorSubcoreMesh(core_axis_name="c", subcore_axis_name="s")

    @functools.partial(
        pl.kernel,
        out_type=jax.ShapeDtypeStruct((S * 2 * H,), jnp.float32),
        mesh=mesh,
        scratch_types=[
            pltpu.VMEM((N + LANES,), jnp.int32),
            pltpu.VMEM((C, H), jnp.float32),
            pltpu.VMEM((C, H), jnp.float32),
            pltpu.VMEM((2 * H,), jnp.float32),
            pltpu.SemaphoreType.DMA,
            pltpu.SemaphoreType.DMA,
        ],
    )
    def k(x_hbm, batch_hbm, out_hbm, batch_v, buf_a, buf_b, orow_v,
          sem_a, sem_b):
        wid = lax.axis_index("s") * info.num_cores + lax.axis_index("c")
        seg0 = wid * SPW
        pltpu.sync_copy(batch_hbm, batch_v.at[pl.ds(0, N)])

        def lower_bound(t):
            def bs_body(_, lh):
                lo, hi = lh
                mid = lax.shift_right_logical(lo + hi, 1)
                val = batch_v[pl.ds(mid, LANES)][0]
                pred = val < t
                return jnp.where(pred, mid + 1, lo), jnp.where(pred, hi, mid)

            lo, _ = lax.fori_loop(
                0, 17, bs_body, (jnp.int32(0), jnp.int32(N))
            )
            return lo

        bounds = [lower_bound(seg0 + kk) for kk in range(SPW + 1)]

        for kseg in range(SPW):
            r0 = bounds[kseg]
            r1 = bounds[kseg + 1]
            cnt = r1 - r0
            base_a = lax.bitwise_and(r0, jnp.int32(~7))
            nchunks = lax.shift_right_logical(r1 - base_a + (C - 1), CSH)

            zero = jnp.zeros((LANES,), jnp.float32)
            fmin = jnp.full((LANES,), _FMIN, jnp.float32)
            for f in range(F):
                orow_v[pl.ds(f * LANES, LANES)] = zero
                orow_v[pl.ds(H + f * LANES, LANES)] = fmin

            def start_dma(i, buf, sem, base_a=base_a):
                start = base_a + i * C
                start_cl = pl.multiple_of(jnp.minimum(start, N - C), 8)
                pltpu.async_copy(x_hbm.at[pl.ds(start_cl, C)], buf, sem)

            def wait_dma(buf, sem):
                pltpu.make_async_copy(x_hbm.at[pl.ds(0, C)], buf, sem).wait()

            def compute_acc(i, buf, r0=r0, r1=r1, base_a=base_a):
                start = base_a + i * C
                sh = start - jnp.minimum(start, N - C)
                jlo = jnp.maximum(r0 - start, 0) + sh
                jhi = jnp.minimum(C, r1 - start) + sh
                s = [orow_v[pl.ds(f * LANES, LANES)] for f in range(F)]
                m = [orow_v[pl.ds(H + f * LANES, LANES)] for f in range(F)]

                def row_body(j, car):
                    s, m = car
                    new_s, new_m = [], []
                    for f in range(F):
                        v = buf[j, pl.ds(f * LANES, LANES)]
                        new_s.append(s[f] + v)
                        new_m.append(jnp.maximum(m[f], v))
                    return tuple(new_s), tuple(new_m)

                s, m = lax.fori_loop(jlo, jhi, row_body, (tuple(s), tuple(m)))
                for f in range(F):
                    orow_v[pl.ds(f * LANES, LANES)] = s[f]
                    orow_v[pl.ds(H + f * LANES, LANES)] = m[f]

            @pl.when(nchunks >= 1)
            def _():
                start_dma(0, buf_a, sem_a)

            def pair_body(p, carry, nchunks=nchunks):
                i0 = 2 * p

                start_dma(i0 + 1, buf_b, sem_b)
                wait_dma(buf_a, sem_a)
                compute_acc(i0, buf_a)

                @pl.when(i0 + 2 < nchunks)
                def _():
                    start_dma(i0 + 2, buf_a, sem_a)

                wait_dma(buf_b, sem_b)
                compute_acc(i0 + 1, buf_b)
                return carry

            npairs = lax.shift_right_logical(nchunks, 1)
            lax.fori_loop(0, npairs, pair_body, jnp.int32(0))

            @pl.when(lax.bitwise_and(nchunks, 1) == 1)
            def _(nchunks=nchunks):
                wait_dma(buf_a, sem_a)
                compute_acc(nchunks - 1, buf_a)

            cntf = lax.broadcast_in_dim(cnt, (LANES,), ()).astype(jnp.float32)
            scale = 1.0 / jnp.maximum(cntf, 1.0)
            nonempty = jnp.minimum(cntf, 1.0)
            for f in range(F):
                orow_v[pl.ds(f * LANES, LANES)] = (
                    orow_v[pl.ds(f * LANES, LANES)] * scale
                )
                orow_v[pl.ds(H + f * LANES, LANES)] = (
                    orow_v[pl.ds(H + f * LANES, LANES)] * nonempty
                )
            pltpu.sync_copy(
                orow_v, out_hbm.at[pl.ds((seg0 + kseg) * 2 * H, 2 * H)]
            )

    return k


def kernel(x, batch):
    N, H = x.shape
    out = _make_sc_kernel(N, H, NUM_SEGS, CHUNK)(x, batch)
    return out.reshape(NUM_SEGS, 2 * H)


# speedup vs baseline: 15.6782x; 1.1679x over previous
import functools

import jax
import jax.numpy as jnp
from jax import lax
from jax.experimental import pallas as pl
from jax.experimental.pallas import tpu as pltpu
from jax.experimental.pallas import tpu_sc as plsc

NUM_SEGS = 128
LANES = 16
CHUNK = 128
_FMIN = float(jnp.finfo(jnp.float32).min)


@functools.cache
def _make_sc_kernel(N, H, S, C):
    info = plsc.get_sparse_core_info()
    NW = info.num_cores * info.num_subcores
    assert S % NW == 0 and H % LANES == 0 and N % LANES == 0
    SPW = S // NW
    F = H // LANES
    CSH = C.bit_length() - 1
    mesh = plsc.VectorSubcoreMesh(core_axis_name="c", subcore_axis_name="s")

    @functools.partial(
        pl.kernel,
        out_type=jax.ShapeDtypeStruct((S * 2 * H,), jnp.float32),
        mesh=mesh,
        scratch_types=[
            pltpu.VMEM((N + LANES,), jnp.int32),
            pltpu.VMEM((C, H), jnp.float32),
            pltpu.VMEM((C, H), jnp.float32),
            pltpu.VMEM((SPW * 2 * H,), jnp.float32),
            pltpu.SemaphoreType.DMA,
            pltpu.SemaphoreType.DMA,
        ],
    )
    def k(x_hbm, batch_hbm, out_hbm, batch_v, buf_a, buf_b, acc_v,
          sem_a, sem_b):
        wid = lax.axis_index("s") * info.num_cores + lax.axis_index("c")
        seg0 = wid * SPW
        pltpu.sync_copy(batch_hbm, batch_v.at[pl.ds(0, N)])

        def bs_body(_, lohis):
            new = []
            for t, (lo, hi) in enumerate(lohis):
                mid = lax.shift_right_logical(lo + hi, 1)
                val = batch_v[pl.ds(mid, LANES)][0]
                pred = val < seg0 + t
                new.append(
                    (jnp.where(pred, mid + 1, lo), jnp.where(pred, hi, mid))
                )
            return tuple(new)

        init = ((jnp.int32(0), jnp.int32(N)),) * (SPW + 1)
        bounds = [lh[0] for lh in lax.fori_loop(0, 17, bs_body, init)]

        zero = jnp.zeros((LANES,), jnp.float32)
        fmin = jnp.full((LANES,), _FMIN, jnp.float32)
        for kseg in range(SPW):
            for f in range(F):
                acc_v[pl.ds(kseg * 2 * H + f * LANES, LANES)] = zero
                acc_v[pl.ds(kseg * 2 * H + H + f * LANES, LANES)] = fmin

        r_begin = bounds[0]
        r_end = bounds[SPW]
        base_a = lax.bitwise_and(r_begin, jnp.int32(~7))
        nchunks = lax.shift_right_logical(r_end - base_a + (C - 1), CSH)

        def start_dma(i, buf, sem):
            start = base_a + i * C
            start_cl = pl.multiple_of(jnp.minimum(start, N - C), 8)
            pltpu.async_copy(x_hbm.at[pl.ds(start_cl, C)], buf, sem)

        def wait_dma(buf, sem):
            pltpu.make_async_copy(x_hbm.at[pl.ds(0, C)], buf, sem).wait()

        def compute_acc(i, buf):
            start = base_a + i * C
            sh = start - jnp.minimum(start, N - C)
            for kseg in range(SPW):
                jlo = jnp.maximum(bounds[kseg] - start, 0) + sh
                jhi = jnp.minimum(C, bounds[kseg + 1] - start) + sh

                @pl.when(jlo < jhi)
                def _(jlo=jlo, jhi=jhi, kseg=kseg):
                    ab = kseg * 2 * H
                    s = [
                        acc_v[pl.ds(ab + f * LANES, LANES)] for f in range(F)
                    ]
                    m = [
                        acc_v[pl.ds(ab + H + f * LANES, LANES)]
                        for f in range(F)
                    ]

                    def row_body(j, car):
                        s, m = car
                        new_s, new_m = [], []
                        for f in range(F):
                            v = buf[j, pl.ds(f * LANES, LANES)]
                            new_s.append(s[f] + v)
                            new_m.append(jnp.maximum(m[f], v))
                        return tuple(new_s), tuple(new_m)

                    s, m = lax.fori_loop(
                        jlo, jhi, row_body, (tuple(s), tuple(m))
                    )
                    for f in range(F):
                        acc_v[pl.ds(ab + f * LANES, LANES)] = s[f]
                        acc_v[pl.ds(ab + H + f * LANES, LANES)] = m[f]

        @pl.when(nchunks >= 1)
        def _():
            start_dma(0, buf_a, sem_a)

        def pair_body(p, carry):
            i0 = 2 * p
            start_dma(i0 + 1, buf_b, sem_b)
            wait_dma(buf_a, sem_a)
            compute_acc(i0, buf_a)

            @pl.when(i0 + 2 < nchunks)
            def _():
                start_dma(i0 + 2, buf_a, sem_a)

            wait_dma(buf_b, sem_b)
            compute_acc(i0 + 1, buf_b)
            return carry

        npairs = lax.shift_right_logical(nchunks, 1)
        lax.fori_loop(0, npairs, pair_body, jnp.int32(0))

        @pl.when(lax.bitwise_and(nchunks, 1) == 1)
        def _():
            wait_dma(buf_a, sem_a)
            compute_acc(nchunks - 1, buf_a)

        for kseg in range(SPW):
            ab = kseg * 2 * H
            cnt = bounds[kseg + 1] - bounds[kseg]
            cntf = lax.broadcast_in_dim(cnt, (LANES,), ()).astype(jnp.float32)
            scale = 1.0 / jnp.maximum(cntf, 1.0)
            nonempty = jnp.minimum(cntf, 1.0)
            for f in range(F):
                acc_v[pl.ds(ab + f * LANES, LANES)] = (
                    acc_v[pl.ds(ab + f * LANES, LANES)] * scale
                )
                acc_v[pl.ds(ab + H + f * LANES, LANES)] = (
                    acc_v[pl.ds(ab + H + f * LANES, LANES)] * nonempty
                )
        pltpu.sync_copy(
            acc_v, out_hbm.at[pl.ds(seg0 * 2 * H, SPW * 2 * H)]
        )

    return k


def kernel(x, batch):
    N, H = x.shape
    out = _make_sc_kernel(N, H, NUM_SEGS, CHUNK)(x, batch)
    return out.reshape(NUM_SEGS, 2 * H)
